# baseline (speedup 1.0000x reference)
import functools

import jax
import jax.numpy as jnp
from jax import lax
from jax.experimental import pallas as pl
from jax.experimental.pallas import tpu as pltpu
from jax.experimental.pallas import tpu_sc as plsc

_LANES = 128


def _log_softmax_table_kernel(u_ref, it_ref, out_ref):
    u = u_ref[...]
    it = it_ref[...]
    util = lax.dot_general(u, it, (((1,), (1,)), ((), ())),
                           preferred_element_type=jnp.float32)
    m = jnp.max(util, axis=1, keepdims=True)
    e = jnp.exp(util - m)
    lse = m + jnp.log(jnp.sum(e, axis=1, keepdims=True))
    out_ref[...] = util - lse


@functools.cache
def _make_expand(U, I, B):
    info = plsc.get_sparse_core_info()
    NC, NS = info.num_cores, info.num_subcores
    NW = NC * NS
    SPW = B // NW
    assert B % NW == 0
    C = 32
    n_chunks = SPW // C
    assert SPW % C == 0
    mesh = plsc.VectorSubcoreMesh(core_axis_name="c", subcore_axis_name="s")

    @functools.partial(
        pl.kernel, mesh=mesh,
        out_type=jax.ShapeDtypeStruct((B, I), jnp.float32),
        compiler_params=pltpu.CompilerParams(needs_layout_passes=False),
        scratch_types=[
            pltpu.VMEM((SPW,), jnp.int32),
            pltpu.SemaphoreType.DMA,
            pltpu.SemaphoreType.DMA,
        ],
    )
    def expand(table_hbm, idx_hbm, out_hbm, idx_v, semi, semw):
        wid = lax.axis_index("s") * NC + lax.axis_index("c")
        s0 = pl.multiple_of(wid * SPW, SPW)
        pltpu.async_copy(idx_hbm.at[pl.ds(s0, SPW)], idx_v, semi).wait()

        def chunk(c, carry):
            base = c * C
            for k in range(C // 16):
                u16 = idx_v[pl.ds(base + 16 * k, 16)]
                for j in range(16):
                    r = 16 * k + j
                    pltpu.async_copy(table_hbm.at[u16[j]],
                                     out_hbm.at[s0 + base + r], semw)
            return carry

        lax.fori_loop(0, n_chunks, chunk, 0)
        def drainer(c, carry):
            for r in range(C):
                pltpu.make_async_copy(table_hbm.at[0],
                                      out_hbm.at[0], semw).wait()
            return carry

        lax.fori_loop(0, n_chunks, drainer, 0)

    return expand


def kernel(user_latent_value, item_latent_value, user_idx):
    S, U, D = user_latent_value.shape
    I = item_latent_value.shape[1]
    B = user_idx.shape[0]
    Upad = (U + _LANES - 1) // _LANES * _LANES
    u2 = user_latent_value.reshape(U, D)
    u2 = jnp.pad(u2, ((0, Upad - U), (0, 0)))
    it2 = item_latent_value.reshape(I, D)
    table = pl.pallas_call(
        _log_softmax_table_kernel,
        out_shape=jax.ShapeDtypeStruct((Upad, I), jnp.float32),
    )(u2, it2)
    out2 = _make_expand(U, I, B)(table, user_idx.astype(jnp.int32))
    return out2[None]


# speedup vs baseline: 16.6739x; 16.6739x over previous
import functools

import jax
import jax.numpy as jnp
from jax import lax
from jax.experimental import pallas as pl
from jax.experimental.pallas import tpu as pltpu
from jax.experimental.pallas import tpu_sc as plsc

_LANES = 128


def _log_softmax_table_kernel(I, u_ref, it_ref, out_ref):
    u = u_ref[...]
    it = it_ref[...]
    util = lax.dot_general(u, it, (((1,), (1,)), ((), ())),
                           preferred_element_type=jnp.float32)
    col = lax.broadcasted_iota(jnp.int32, util.shape, 1)
    util = jnp.where(col < I, util, -jnp.inf)
    m = jnp.max(util, axis=1, keepdims=True)
    e = jnp.exp(util - m)
    lse = m + jnp.log(jnp.sum(e, axis=1, keepdims=True))
    out_ref[...] = util - lse


@functools.cache
def _make_expand(U, I, Ipad, B):
    info = plsc.get_sparse_core_info()
    NC, NS = info.num_cores, info.num_subcores
    NW = NC * NS
    SPW = B // NW
    assert B % NW == 0
    C = 32
    n_chunks = SPW // C
    assert SPW % C == 0 and n_chunks % 2 == 0
    MAIN = Ipad - _LANES
    TAIL = I - MAIN
    assert 0 < TAIL <= _LANES and MAIN % _LANES == 0
    mesh = plsc.VectorSubcoreMesh(core_axis_name="c", subcore_axis_name="s")

    @functools.partial(
        pl.kernel, mesh=mesh,
        out_type=jax.ShapeDtypeStruct((B, I), jnp.float32),
        compiler_params=pltpu.CompilerParams(needs_layout_passes=False),
        scratch_types=[
            pltpu.VMEM((SPW,), jnp.int32),
            pltpu.VMEM((C, Ipad), jnp.float32),
            pltpu.VMEM((C, Ipad), jnp.float32),
            pltpu.VMEM((C, TAIL), jnp.float32),
            pltpu.VMEM((C, TAIL), jnp.float32),
            pltpu.SemaphoreType.DMA,
            pltpu.SemaphoreType.DMA,
            pltpu.SemaphoreType.DMA,
        ],
    )
    def expand(table_hbm, idx_hbm, out_hbm,
               idx_v, bufA, bufB, tailA, tailB, semi, semsA, semsB):
        wid = lax.axis_index("s") * NC + lax.axis_index("c")
        s0 = pl.multiple_of(wid * SPW, SPW)
        pltpu.async_copy(idx_hbm.at[pl.ds(s0, SPW)], idx_v, semi).wait()

        def gather(buf, c):
            off = pl.multiple_of(c * C, C)
            return pltpu.async_copy(
                table_hbm.at[idx_v.at[pl.ds(off, C)]], buf, semi)

        ktail = lax.iota(jnp.int32, 16) + MAIN // 16 * 0
        tcols = lax.iota(jnp.int32, 16) + (TAIL - 8)
        tmask = lax.iota(jnp.int32, 16) < 8
        zeros16 = jnp.zeros((16,), jnp.int32)

        def repack(buf, tail):
            def body(r, carry):
                for k in range(TAIL // 16):
                    tail[r, pl.ds(16 * k, 16)] = buf[r, pl.ds(MAIN + 16 * k,
                                                              16)]
                x = buf[r, pl.ds(MAIN + TAIL - 8, 16)]
                plsc.store_scatter(tail, [zeros16 + r, tcols], x, mask=tmask)
                return carry

            lax.fori_loop(0, C, body, 0)

        def scatter(buf, tail, sems, c):
            row = pl.multiple_of(s0 + c * C, C)
            pltpu.async_copy(buf.at[:, pl.ds(0, MAIN)],
                             out_hbm.at[pl.ds(row, C), pl.ds(0, MAIN)], sems)
            pltpu.async_copy(tail,
                             out_hbm.at[pl.ds(row, C), pl.ds(MAIN, TAIL)],
                             sems)

        def drain(buf, tail, sems):
            pltpu.make_async_copy(
                buf.at[:, pl.ds(0, MAIN)],
                out_hbm.at[pl.ds(0, C), pl.ds(0, MAIN)], sems).wait()
            pltpu.make_async_copy(
                tail, out_hbm.at[pl.ds(0, C), pl.ds(MAIN, TAIL)], sems).wait()

        def chunk_pair(p, carry):
            c0 = 2 * p
            cpA = gather(bufA, c0)
            cpB = gather(bufB, c0 + 1)
            cpA.wait()
            repack(bufA, tailA)

            @pl.when(p > 0)
            def _():
                drain(bufA, tailA, semsA)

            scatter(bufA, tailA, semsA, c0)
            cpB.wait()
            repack(bufB, tailB)

            @pl.when(p > 0)
            def _():
                drain(bufB, tailB, semsB)

            scatter(bufB, tailB, semsB, c0 + 1)
            return carry

        lax.fori_loop(0, n_chunks // 2, chunk_pair, 0)
        drain(bufA, tailA, semsA)
        drain(bufB, tailB, semsB)

    return expand


def kernel(user_latent_value, item_latent_value, user_idx):
    S, U, D = user_latent_value.shape
    I = item_latent_value.shape[1]
    B = user_idx.shape[0]
    Upad = (U + _LANES - 1) // _LANES * _LANES
    Ipad = (I + _LANES - 1) // _LANES * _LANES
    u2 = user_latent_value.reshape(U, D)
    u2 = jnp.pad(u2, ((0, Upad - U), (0, 0)))
    it2 = item_latent_value.reshape(I, D)
    it2 = jnp.pad(it2, ((0, Ipad - I), (0, 0)))
    table = pl.pallas_call(
        functools.partial(_log_softmax_table_kernel, I),
        out_shape=jax.ShapeDtypeStruct((Upad, Ipad), jnp.float32),
    )(u2, it2)
    out2 = _make_expand(U, I, Ipad, B)(table, user_idx.astype(jnp.int32))
    return out2[None]
